# baseline (speedup 1.0000x reference)
import jax
import jax.numpy as jnp
from jax import lax
from jax.experimental import pallas as pl
from jax.experimental.pallas import tpu as pltpu
from jax.experimental.pallas import tpu_sc as plsc

D = 64
DW = D // 2
NSEG = 16
SEQ = 512
NC = 2
NS = 16
NW = NC * NS
CHUNK = 256
NSTREAM = CHUNK // 128
LANES = 16
NBUF = 2


HIMASK = -65536


def _as_f32(x):
    return lax.bitcast_convert_type(x, jnp.float32)


def _as_i32(x):
    return lax.bitcast_convert_type(x, jnp.int32)


def _round_bf16_bits(x):
    bits = lax.bitcast_convert_type(x, jnp.int32)
    lsb = lax.shift_right_logical(bits, 16) & 1
    return lax.shift_right_logical(bits + 0x7FFF + lsb, 16)


def _body(ids_hbm, segs_hbm, tok_hbm, segev_hbm, segod_hbm, posev_hbm,
          posod_hbm, out_hbm, comb_hbm,
          posev_v, posod_v, segev_v, segod_v, buildp_v,
          idx_tok_v, idx_comb_v, seg_v, tok_v, comb_v,
          sem_tok, sem_comb, sem_out):
    c = lax.axis_index("c")
    s = lax.axis_index("s")
    wid = c * NS + s
    batch, seq = ids_hbm.shape
    n_tokens = batch * seq
    per_w = n_tokens // NW
    nchunk = per_w // CHUNK
    core_off = c * (NSEG * SEQ)
    iota = lax.iota(jnp.int32, LANES)

    pltpu.sync_copy(posev_hbm, posev_v)
    pltpu.sync_copy(posod_hbm, posod_v)
    pltpu.sync_copy(segev_hbm.at[s], segev_v)
    pltpu.sync_copy(segod_hbm.at[s], segod_v)

    def build_row(r, carry):
        for j in range(DW // LANES):
            sl = pl.ds(j * LANES, LANES)
            ev = _round_bf16_bits(posev_v[r, sl] + segev_v[sl])
            od = _round_bf16_bits(posod_v[r, sl] + segod_v[sl])
            buildp_v[r, sl] = ev | lax.shift_left(od, 16)
        return carry

    lax.fori_loop(0, SEQ, build_row, 0)
    pltpu.sync_copy(buildp_v, comb_hbm.at[pl.ds(core_off + s * SEQ, SEQ)])
    plsc.subcore_barrier()

    wbase = wid * per_w

    def gather_descs(b):
        descs = []
        for j in range(NSTREAM):
            dst_sl = pl.ds(j * 128, 128)
            descs.append(pltpu.make_async_copy(
                tok_hbm.at[idx_tok_v[b].at[j]], tok_v[b].at[dst_sl],
                sem_tok[b]))
            descs.append(pltpu.make_async_copy(
                comb_hbm.at[idx_comb_v[b].at[j]], comb_v[b].at[dst_sl],
                sem_comb[b]))
        return descs

    def out_slot(i):
        base = wbase + i * CHUNK
        return out_hbm.at[base // SEQ, pl.ds(lax.rem(base, SEQ), CHUNK)]

    def start(i, b):
        base = wbase + i * CHUNK
        row = base // SEQ
        col = lax.rem(base, SEQ)

        @pl.when(i >= NBUF)
        def _():
            pltpu.make_async_copy(tok_v[b], out_slot(i), sem_out[b]).wait()

        for j in range(NSTREAM):
            pltpu.sync_copy(ids_hbm.at[row, pl.ds(col + j * 128, 128)],
                            idx_tok_v[b].at[j])
        pltpu.sync_copy(segs_hbm.at[row, pl.ds(col, CHUNK)], seg_v[b])
        for g in range(CHUNK // LANES):
            j, off = divmod(g * LANES, 128)
            idx_comb_v[b][j, pl.ds(off, LANES)] = (
                seg_v[b][pl.ds(g * LANES, LANES)] * SEQ
                + (col + g * LANES + core_off) + iota)
        for d in gather_descs(b):
            d.start()

    def finish(i, b):
        for d in gather_descs(b):
            d.wait()

        def add_row(r, carry):
            for j in range(DW // LANES):
                sl = pl.ds(j * LANES, LANES)
                t = tok_v[b][r, sl]
                cw = comb_v[b][r, sl]
                lo = (_as_f32(lax.shift_left(t, 16))
                      + _as_f32(lax.shift_left(cw, 16)))
                hi = _as_f32(t & HIMASK) + _as_f32(cw & HIMASK)
                lo_b = lax.shift_right_logical(_as_i32(lo) + 0x8000, 16)
                hi_b = (_as_i32(hi) + 0x8000) & HIMASK
                tok_v[b][r, sl] = lo_b | hi_b
            return carry

        lax.fori_loop(0, CHUNK, add_row, 0)
        pltpu.async_copy(tok_v[b], out_slot(i), sem_out[b])

    for b in range(NBUF):
        start(b, b)

    def pair_step(g, carry):
        for b in range(NBUF):
            i = g * NBUF + b
            finish(i, b)

            @pl.when(i + NBUF < nchunk)
            def _():
                start(i + NBUF, b)
        return carry

    lax.fori_loop(0, nchunk // NBUF, pair_step, 0)
    for b in range(NBUF):
        pltpu.make_async_copy(tok_v[b], out_slot(0), sem_out[b]).wait()


def kernel(input_ids, segment_ids, token_embedding_matrix,
           segment_embedding_matrix, position_embedding_matrix):
    batch, seq = input_ids.shape
    segf = segment_embedding_matrix.astype(jnp.float32)
    posf = position_embedding_matrix.astype(jnp.float32)
    tok_packed = lax.bitcast_convert_type(
        token_embedding_matrix.astype(jnp.bfloat16).reshape(-1, DW, 2),
        jnp.int32)

    mesh = plsc.VectorSubcoreMesh(core_axis_name="c", subcore_axis_name="s",
                                  num_cores=NC, num_subcores=NS)
    run = pl.kernel(
        _body,
        out_type=(
            jax.ShapeDtypeStruct((batch, seq, DW), jnp.int32),
            jax.ShapeDtypeStruct((NC * NSEG * SEQ, DW), jnp.int32),
        ),
        mesh=mesh,
        compiler_params=pltpu.CompilerParams(use_tc_tiling_on_sc=False),
        scratch_types=(
            pltpu.VMEM((SEQ, DW), jnp.float32),
            pltpu.VMEM((SEQ, DW), jnp.float32),
            pltpu.VMEM((DW,), jnp.float32),
            pltpu.VMEM((DW,), jnp.float32),
            pltpu.VMEM((SEQ, DW), jnp.int32),
            [pltpu.VMEM((NSTREAM, 128), jnp.int32)] * NBUF,
            [pltpu.VMEM((NSTREAM, 128), jnp.int32)] * NBUF,
            [pltpu.VMEM((CHUNK,), jnp.int32)] * NBUF,
            [pltpu.VMEM((CHUNK, DW), jnp.int32)] * NBUF,
            [pltpu.VMEM((CHUNK, DW), jnp.int32)] * NBUF,
            [pltpu.SemaphoreType.DMA] * NBUF,
            [pltpu.SemaphoreType.DMA] * NBUF,
            [pltpu.SemaphoreType.DMA] * NBUF,
        ),
    )
    out, _ = run(input_ids.astype(jnp.int32), segment_ids.astype(jnp.int32),
                 tok_packed,
                 segf[:, 0::2], segf[:, 1::2], posf[:, 0::2], posf[:, 1::2])
    out16 = lax.bitcast_convert_type(out, jnp.bfloat16)
    return out16.reshape(batch, seq, D).astype(jnp.float32)


# speedup vs baseline: 1.7735x; 1.7735x over previous
import jax
import jax.numpy as jnp
from jax import lax
from jax.experimental import pallas as pl
from jax.experimental.pallas import tpu as pltpu
from jax.experimental.pallas import tpu_sc as plsc

D = 64
NSEG = 16
SEQ = 512
NC = 2
NS = 16
NW = NC * NS
CHUNK = 256
NSTREAM = CHUNK // 128
LANES = 16
NBUF = 2


def _body(ids_hbm, segs_hbm, tok_hbm, comb_hbm, out_hbm,
          idx_tok_v, idx_comb_v, seg_v, tok_v, comb_v,
          sem_tok, sem_comb, sem_out):
    c = lax.axis_index("c")
    s = lax.axis_index("s")
    wid = c * NS + s
    batch, seq = ids_hbm.shape
    n_tokens = batch * seq
    per_w = n_tokens // NW
    nchunk = per_w // CHUNK
    iota = lax.iota(jnp.int32, LANES)
    wbase = wid * per_w

    def gather_descs(b):
        descs = []
        for j in range(NSTREAM):
            dst_sl = pl.ds(j * 128, 128)
            descs.append(pltpu.make_async_copy(
                tok_hbm.at[idx_tok_v[b].at[j]], tok_v[b].at[dst_sl],
                sem_tok[b]))
            descs.append(pltpu.make_async_copy(
                comb_hbm.at[idx_comb_v[b].at[j]], comb_v[b].at[dst_sl],
                sem_comb[b]))
        return descs

    def out_slot(i):
        base = wbase + i * CHUNK
        return out_hbm.at[base // SEQ, pl.ds(lax.rem(base, SEQ), CHUNK)]

    def start(i, b):
        base = wbase + i * CHUNK
        row = base // SEQ
        col = lax.rem(base, SEQ)

        @pl.when(i >= NBUF)
        def _():
            pltpu.make_async_copy(tok_v[b], out_slot(i), sem_out[b]).wait()

        for j in range(NSTREAM):
            pltpu.sync_copy(ids_hbm.at[row, pl.ds(col + j * 128, 128)],
                            idx_tok_v[b].at[j])
        pltpu.sync_copy(segs_hbm.at[row, pl.ds(col, CHUNK)], seg_v[b])
        for g in range(CHUNK // LANES):
            j, off = divmod(g * LANES, 128)
            idx_comb_v[b][j, pl.ds(off, LANES)] = (
                seg_v[b][pl.ds(g * LANES, LANES)] * SEQ
                + (col + g * LANES) + iota)
        for d in gather_descs(b):
            d.start()

    def finish(i, b):
        for d in gather_descs(b):
            d.wait()

        def add_row(r, carry):
            for j in range(D // (2 * LANES)):
                sl = pl.ds(j * 2 * LANES, 2 * LANES)
                tok_v[b][r, sl] = tok_v[b][r, sl] + comb_v[b][r, sl]
            return carry

        lax.fori_loop(0, CHUNK, add_row, 0)
        pltpu.async_copy(tok_v[b], out_slot(i), sem_out[b])

    for b in range(NBUF):
        start(b, b)

    def pair_step(g, carry):
        for b in range(NBUF):
            i = g * NBUF + b
            finish(i, b)

            @pl.when(i + NBUF < nchunk)
            def _():
                start(i + NBUF, b)
        return carry

    lax.fori_loop(0, nchunk // NBUF, pair_step, 0)
    for b in range(NBUF):
        pltpu.make_async_copy(tok_v[b], out_slot(0), sem_out[b]).wait()


def kernel(input_ids, segment_ids, token_embedding_matrix,
           segment_embedding_matrix, position_embedding_matrix):
    batch, seq = input_ids.shape
    comb16 = (segment_embedding_matrix.astype(jnp.float32)[:, None, :]
              + position_embedding_matrix.astype(jnp.float32)[None, :, :]
              ).astype(jnp.bfloat16).reshape(NSEG * SEQ, D)

    mesh = plsc.VectorSubcoreMesh(core_axis_name="c", subcore_axis_name="s",
                                  num_cores=NC, num_subcores=NS)
    run = pl.kernel(
        _body,
        out_type=jax.ShapeDtypeStruct((batch, seq, D), jnp.bfloat16),
        mesh=mesh,
        compiler_params=pltpu.CompilerParams(use_tc_tiling_on_sc=False),
        scratch_types=(
            [pltpu.VMEM((NSTREAM, 128), jnp.int32)] * NBUF,
            [pltpu.VMEM((NSTREAM, 128), jnp.int32)] * NBUF,
            [pltpu.VMEM((CHUNK,), jnp.int32)] * NBUF,
            [pltpu.VMEM((CHUNK, D), jnp.bfloat16)] * NBUF,
            [pltpu.VMEM((CHUNK, D), jnp.bfloat16)] * NBUF,
            [pltpu.SemaphoreType.DMA] * NBUF,
            [pltpu.SemaphoreType.DMA] * NBUF,
            [pltpu.SemaphoreType.DMA] * NBUF,
        ),
    )
    out = run(input_ids.astype(jnp.int32), segment_ids.astype(jnp.int32),
              token_embedding_matrix.astype(jnp.bfloat16), comb16)
    return out.astype(jnp.float32)


# speedup vs baseline: 2.4476x; 1.3801x over previous
import jax
import jax.numpy as jnp
from jax import lax
from jax.experimental import pallas as pl
from jax.experimental.pallas import tpu as pltpu
from jax.experimental.pallas import tpu_sc as plsc

D = 64
NSEG = 16
SEQ = 512
NC = 2
NS = 16
NW = NC * NS
CHUNK = 256
NSTREAM = CHUNK // 128
LANES = 16
NBUF = 2


def _body(ids_hbm, segs_hbm, tok_hbm, comb_hbm, out_hbm,
          ids_v, cidx_v, tok_v, comb_v,
          sem_tok, sem_comb, sem_out):
    c = lax.axis_index("c")
    s = lax.axis_index("s")
    wid = c * NS + s
    batch, seq = ids_hbm.shape
    n_tokens = batch * seq
    per_w = n_tokens // NW
    nchunk = per_w // CHUNK
    rows_w = per_w // SEQ
    iota = lax.iota(jnp.int32, LANES)
    wbase = wid * per_w
    wrow = wid * rows_w

    for r in range(per_w // SEQ):
        pltpu.sync_copy(ids_hbm.at[wrow + r], ids_v.at[pl.ds(r * SEQ, SEQ)])
        pltpu.sync_copy(segs_hbm.at[wrow + r], cidx_v.at[pl.ds(r * SEQ, SEQ)])

    def idx_group(g, carry):
        sl = pl.ds(g * LANES, LANES)
        cidx_v[sl] = (cidx_v[sl] * SEQ
                      + (lax.rem(g, SEQ // LANES) * LANES) + iota)
        return carry

    lax.fori_loop(0, per_w // LANES, idx_group, 0)

    def gather_descs(b, i):
        local = i * CHUNK
        descs = []
        for j in range(NSTREAM):
            sl = pl.ds(local + j * 128, 128)
            dst_sl = pl.ds(j * 128, 128)
            descs.append(pltpu.make_async_copy(
                tok_hbm.at[ids_v.at[sl]], tok_v[b].at[dst_sl], sem_tok[b]))
            descs.append(pltpu.make_async_copy(
                comb_hbm.at[cidx_v.at[sl]], comb_v[b].at[dst_sl],
                sem_comb[b]))
        return descs

    def out_slot(i):
        base = wbase + i * CHUNK
        return out_hbm.at[base // SEQ, pl.ds(lax.rem(base, SEQ), CHUNK)]

    def start(i, b):
        @pl.when(i >= NBUF)
        def _():
            pltpu.make_async_copy(tok_v[b], out_slot(i), sem_out[b]).wait()

        for d in gather_descs(b, i):
            d.start()

    def finish(i, b):
        for d in gather_descs(b, i):
            d.wait()

        def add_row(r, carry):
            for j in range(D // LANES):
                sl = pl.ds(j * LANES, LANES)
                tok_v[b][r, sl] = tok_v[b][r, sl] + comb_v[b][r, sl]
            return carry

        lax.fori_loop(0, CHUNK, add_row, 0)
        pltpu.async_copy(tok_v[b], out_slot(i), sem_out[b])

    for b in range(NBUF):
        start(b, b)

    def pair_step(g, carry):
        for b in range(NBUF):
            i = g * NBUF + b
            finish(i, b)

            @pl.when(i + NBUF < nchunk)
            def _():
                start(i + NBUF, b)
        return carry

    lax.fori_loop(0, nchunk // NBUF, pair_step, 0)
    for b in range(NBUF):
        pltpu.make_async_copy(tok_v[b], out_slot(0), sem_out[b]).wait()


def kernel(input_ids, segment_ids, token_embedding_matrix,
           segment_embedding_matrix, position_embedding_matrix):
    batch, seq = input_ids.shape
    comb = (segment_embedding_matrix.astype(jnp.float32)[:, None, :]
            + position_embedding_matrix.astype(jnp.float32)[None, :, :]
            ).reshape(NSEG * SEQ, D)
    per_w = batch * seq // NW

    mesh = plsc.VectorSubcoreMesh(core_axis_name="c", subcore_axis_name="s",
                                  num_cores=NC, num_subcores=NS)
    run = pl.kernel(
        _body,
        out_type=jax.ShapeDtypeStruct((batch, seq, D), jnp.float32),
        mesh=mesh,
        compiler_params=pltpu.CompilerParams(use_tc_tiling_on_sc=False),
        scratch_types=(
            pltpu.VMEM((per_w,), jnp.int32),
            pltpu.VMEM((per_w,), jnp.int32),
            [pltpu.VMEM((CHUNK, D), jnp.float32)] * NBUF,
            [pltpu.VMEM((CHUNK, D), jnp.float32)] * NBUF,
            [pltpu.SemaphoreType.DMA] * NBUF,
            [pltpu.SemaphoreType.DMA] * NBUF,
            [pltpu.SemaphoreType.DMA] * NBUF,
        ),
    )
    out = run(input_ids.astype(jnp.int32), segment_ids.astype(jnp.int32),
              token_embedding_matrix.astype(jnp.float32), comb)
    return out


# speedup vs baseline: 2.4564x; 1.0036x over previous
import jax
import jax.numpy as jnp
from jax import lax
from jax.experimental import pallas as pl
from jax.experimental.pallas import tpu as pltpu
from jax.experimental.pallas import tpu_sc as plsc

D = 64
NSEG = 16
SEQ = 512
NC = 2
NS = 16
NW = NC * NS
CHUNK = 128
NSTREAM = CHUNK // 128
LANES = 16
NBUF = 4


def _body(ids_hbm, segs_hbm, tok_hbm, comb_hbm, out_hbm,
          ids_v, cidx_v, tok_v, comb_v,
          sem_tok, sem_comb, sem_out):
    c = lax.axis_index("c")
    s = lax.axis_index("s")
    wid = c * NS + s
    batch, seq = ids_hbm.shape
    n_tokens = batch * seq
    per_w = n_tokens // NW
    nchunk = per_w // CHUNK
    rows_w = per_w // SEQ
    iota = lax.iota(jnp.int32, LANES)
    wbase = wid * per_w
    wrow = wid * rows_w

    for r in range(per_w // SEQ):
        pltpu.sync_copy(ids_hbm.at[wrow + r], ids_v.at[pl.ds(r * SEQ, SEQ)])
        pltpu.sync_copy(segs_hbm.at[wrow + r], cidx_v.at[pl.ds(r * SEQ, SEQ)])

    def idx_group(g, carry):
        sl = pl.ds(g * LANES, LANES)
        cidx_v[sl] = (cidx_v[sl] * SEQ
                      + (lax.rem(g, SEQ // LANES) * LANES) + iota)
        return carry

    lax.fori_loop(0, per_w // LANES, idx_group, 0)

    def gather_descs(b, i):
        local = i * CHUNK
        descs = []
        for j in range(NSTREAM):
            sl = pl.ds(local + j * 128, 128)
            dst_sl = pl.ds(j * 128, 128)
            descs.append(pltpu.make_async_copy(
                tok_hbm.at[ids_v.at[sl]], tok_v[b].at[dst_sl], sem_tok[b]))
            descs.append(pltpu.make_async_copy(
                comb_hbm.at[cidx_v.at[sl]], comb_v[b].at[dst_sl],
                sem_comb[b]))
        return descs

    def out_slot(i):
        base = wbase + i * CHUNK
        return out_hbm.at[base // SEQ, pl.ds(lax.rem(base, SEQ), CHUNK)]

    def start(i, b):
        @pl.when(i >= NBUF)
        def _():
            pltpu.make_async_copy(tok_v[b], out_slot(i), sem_out[b]).wait()

        for d in gather_descs(b, i):
            d.start()

    def finish(i, b):
        for d in gather_descs(b, i):
            d.wait()

        def add_row(r, carry):
            for j in range(D // LANES):
                sl = pl.ds(j * LANES, LANES)
                tok_v[b][r, sl] = tok_v[b][r, sl] + comb_v[b][r, sl]
            return carry

        lax.fori_loop(0, CHUNK, add_row, 0)
        pltpu.async_copy(tok_v[b], out_slot(i), sem_out[b])

    for b in range(NBUF):
        start(b, b)

    def pair_step(g, carry):
        for b in range(NBUF):
            i = g * NBUF + b
            finish(i, b)

            @pl.when(i + NBUF < nchunk)
            def _():
                start(i + NBUF, b)
        return carry

    lax.fori_loop(0, nchunk // NBUF, pair_step, 0)
    for b in range(NBUF):
        pltpu.make_async_copy(tok_v[b], out_slot(0), sem_out[b]).wait()


def kernel(input_ids, segment_ids, token_embedding_matrix,
           segment_embedding_matrix, position_embedding_matrix):
    batch, seq = input_ids.shape
    comb = (segment_embedding_matrix.astype(jnp.float32)[:, None, :]
            + position_embedding_matrix.astype(jnp.float32)[None, :, :]
            ).reshape(NSEG * SEQ, D)
    per_w = batch * seq // NW

    mesh = plsc.VectorSubcoreMesh(core_axis_name="c", subcore_axis_name="s",
                                  num_cores=NC, num_subcores=NS)
    run = pl.kernel(
        _body,
        out_type=jax.ShapeDtypeStruct((batch, seq, D), jnp.float32),
        mesh=mesh,
        compiler_params=pltpu.CompilerParams(use_tc_tiling_on_sc=False),
        scratch_types=(
            pltpu.VMEM((per_w,), jnp.int32),
            pltpu.VMEM((per_w,), jnp.int32),
            [pltpu.VMEM((CHUNK, D), jnp.float32)] * NBUF,
            [pltpu.VMEM((CHUNK, D), jnp.float32)] * NBUF,
            [pltpu.SemaphoreType.DMA] * NBUF,
            [pltpu.SemaphoreType.DMA] * NBUF,
            [pltpu.SemaphoreType.DMA] * NBUF,
        ),
    )
    out = run(input_ids.astype(jnp.int32), segment_ids.astype(jnp.int32),
              token_embedding_matrix.astype(jnp.float32), comb)
    return out
